# Initial kernel scaffold; baseline (speedup 1.0000x reference)
import jax
import jax.numpy as jnp
from jax.experimental import pallas as pl


def kernel(users, items, adj_list, adj_relation, entity_emb, user_emb, We1, be1, We2, be2, Wd1, bd1, Wd2, bd2, Wr1, lw1, lb1, Wr2, lw2, lb2):
    raise NotImplementedError("write your pallas kernel here")


# speedup vs baseline: 9.5558x; 9.5558x over previous
import functools

import jax
import jax.numpy as jnp
from jax import lax
from jax.experimental import pallas as pl
from jax.experimental.pallas import tpu as pltpu
from jax.experimental.pallas import tpu_sc as plsc

_N_ENT = 45000
_N = 50000
_K = 16
_R = 4
_B = 4096
_NW = 32
_CPW = 1568
_NP = _NW * _CPW
_G = 8
_CH = _CPW // _G
_NBUF = 4
_TW = 48
_BLK = 1024
_NBLK = _NP // _BLK

_F32 = jnp.float32
_I32 = jnp.int32
_PREC = lax.Precision.HIGHEST


def _leaky(x):
    return jnp.where(x >= 0, x, 0.01 * x)


def _gate_precompute(x, rel, wr):
    wsum = jnp.sum(wr, axis=-1)
    p = lax.dot_general(x, wsum[:, :32], (((1,), (1,)), ((), ())),
                        precision=_PREC, preferred_element_type=_F32)
    q = lax.dot_general(x, wsum[:, 32:], (((1,), (1,)), ((), ())),
                        precision=_PREC, preferred_element_type=_F32)
    pg = jnp.zeros(rel.shape, _F32)
    for r in range(_R):
        pg = pg + jnp.where(rel == r, p[:, r][:, None], 0.0)
    return pg, q


def _tc_encode_body(emb_ref, rel_ref, we1, be1, we2, be2, wd1, bd1, wd2, bd2,
                    wr1, t1_ref, pg_ref, qc_ref, enc_ref, sq_ref):
    i = pl.program_id(0)
    emb = emb_ref[...]
    h = jnp.maximum(jnp.dot(emb, we1[...], precision=_PREC,
                            preferred_element_type=_F32) + be1[...], 0.0)
    enc = jnp.dot(h, we2[...], precision=_PREC,
                  preferred_element_type=_F32) + be2[...]
    enc_ref[...] = enc
    h2 = jnp.maximum(jnp.dot(enc, wd1[...], precision=_PREC,
                             preferred_element_type=_F32) + bd1[...], 0.0)
    dec = jnp.dot(h2, wd2[...], precision=_PREC,
                  preferred_element_type=_F32) + bd2[...]
    d = dec - emb
    rows = i * _BLK + lax.broadcasted_iota(_I32, (_BLK, 1), 0)
    d = d * (rows < _N).astype(_F32)
    blksum = jnp.sum(d * d)

    @pl.when(i == 0)
    def _():
        sq_ref[0, 0] = 0.0

    sq_ref[0, 0] += blksum
    rel = rel_ref[...]
    pg, q = _gate_precompute(enc, rel, wr1[...])
    pg_ref[...] = pg
    qc_ref[...] = rel + 32
    t1_ref[...] = jnp.concatenate(
        [enc, q, jnp.zeros((_BLK, _TW - 36), _F32)], axis=1)


def _tc_combine_body(prev_ref, agg_ref, rel_ref, lw, lb, wr2,
                     x_ref, t2_ref, pg_ref, qc_ref):
    prev = prev_ref[...]
    agg = agg_ref[...]
    w = lw[...]
    comb = (jnp.dot(prev, w[:32, :], precision=_PREC,
                    preferred_element_type=_F32)
            + jnp.dot(agg, w[32:, :], precision=_PREC,
                      preferred_element_type=_F32) + lb[...])
    x = _leaky(comb)
    x_ref[...] = x
    rel = rel_ref[...]
    pg, q = _gate_precompute(x, rel, wr2[...])
    pg_ref[...] = pg
    qc_ref[...] = rel + 32
    t2_ref[...] = jnp.concatenate(
        [x, q, jnp.zeros((_BLK, _TW - 36), _F32)], axis=1)


def _tc_final_body(x1_ref, agg_ref, lw, lb, fin_ref):
    x1 = x1_ref[...]
    w = lw[...]
    comb = (jnp.dot(x1, w[:32, :], precision=_PREC,
                    preferred_element_type=_F32)
            + jnp.dot(agg_ref[...], w[32:, :], precision=_PREC,
                      preferred_element_type=_F32) + lb[...])
    fin_ref[...] = jnp.concatenate([x1, _leaky(comb)], axis=1)


def _tc_score_body(u_ref, v_ref, o_ref):
    o_ref[...] = jnp.sum(u_ref[...] * v_ref[...], axis=1, keepdims=True)


def _full(shape):
    return pl.BlockSpec(shape, lambda i: tuple(0 for _ in shape))


def _rows(width, dtype=_F32):
    del dtype
    return pl.BlockSpec((_BLK, width), lambda i: (i, 0))


@functools.cache
def _mesh():
    return plsc.VectorSubcoreMesh(core_axis_name="c", subcore_axis_name="s")


def _wid():
    return lax.axis_index("s") * 2 + lax.axis_index("c")


def _sc_agg_body(t_hbm, adj_hbm, qc_hbm, pg_hbm, out_hbm,
                 adj_v, qc_v, pg_v, rows_v, out_v,
                 gs0, gs1, gs2, gs3, os0, os1, os2, os3):
    gsems = (gs0, gs1, gs2, gs3)
    osems = (os0, os1, os2, os3)
    w = _wid()
    pltpu.sync_copy(adj_hbm.at[w], adj_v)
    pltpu.sync_copy(qc_hbm.at[w], qc_v)
    pltpu.sync_copy(pg_hbm.at[w], pg_v)
    for b in range(_NBUF):
        pltpu.async_copy(t_hbm.at[adj_v.at[b]], rows_v.at[b], gsems[b])

    def chunk(c, b):
        pltpu.make_async_copy(t_hbm.at[adj_v.at[c]], rows_v.at[b],
                              gsems[b]).wait()
        @pl.when(c >= _NBUF)
        def _():
            pltpu.make_async_copy(
                out_v.at[b], out_hbm.at[w, pl.ds((c - _NBUF) * _G, _G)],
                osems[b]).wait()

        for g in range(_G):
            node = c * _G + g
            rowi = g * _K + lax.iota(_I32, 16)
            qs = plsc.load_gather(rows_v.at[b], [rowi, qc_v[node, :]])
            s = pg_v[node, :] + qs
            gate = (1.0 / _K) / (1.0 + jnp.exp(-s))
            acc0 = jnp.zeros((16,), _F32)
            acc1 = jnp.zeros((16,), _F32)
            for k in range(_K):
                gk = gate[k]
                acc0 = acc0 + gk * rows_v[b, g * _K + k, 0:16]
                acc1 = acc1 + gk * rows_v[b, g * _K + k, 16:32]
            out_v[b, g, 0:16] = acc0
            out_v[b, g, 16:32] = acc1
        pltpu.async_copy(out_v.at[b], out_hbm.at[w, pl.ds(c * _G, _G)],
                         osems[b])

        @pl.when(c + _NBUF < _CH)
        def _():
            pltpu.async_copy(t_hbm.at[adj_v.at[c + _NBUF]], rows_v.at[b],
                             gsems[b])

    def body(i, carry):
        c0 = i * _NBUF
        for b in range(_NBUF):
            chunk(c0 + b, b)
        return carry

    lax.fori_loop(0, _CH // _NBUF, body, 0)
    for b in range(_NBUF):
        c_last = _CH - _NBUF + b
        pltpu.make_async_copy(
            out_v.at[b], out_hbm.at[w, pl.ds(c_last * _G, _G)],
            osems[b]).wait()


@functools.cache
def _sc_agg():
    return pl.kernel(
        _sc_agg_body,
        out_type=jax.ShapeDtypeStruct((_NW, _CPW, 32), _F32),
        mesh=_mesh(),
        compiler_params=pltpu.CompilerParams(needs_layout_passes=False,
                                             use_tc_tiling_on_sc=False),
        scratch_types=[
            pltpu.VMEM((_CH, _G * _K), _I32),
            pltpu.VMEM((_CPW, _K), _I32),
            pltpu.VMEM((_CPW, _K), _F32),
            pltpu.VMEM((_NBUF, _G * _K, _TW), _F32),
            pltpu.VMEM((_NBUF, _G, 32), _F32),
        ] + [pltpu.SemaphoreType.DMA] * (2 * _NBUF),
    )


def _sc_pairs_body(fin_hbm, ui_hbm, ii_hbm, uo_hbm, io_hbm,
                   ui_v, ii_v, ur_v, ir_v, s1, s2):
    w = _wid()
    pltpu.sync_copy(ui_hbm.at[w], ui_v)
    pltpu.sync_copy(ii_hbm.at[w], ii_v)
    cu = pltpu.async_copy(fin_hbm.at[ui_v], ur_v, s1)
    ci = pltpu.async_copy(fin_hbm.at[ii_v], ir_v, s2)
    cu.wait()
    ci.wait()
    pltpu.sync_copy(ur_v, uo_hbm.at[w])
    pltpu.sync_copy(ir_v, io_hbm.at[w])


_PPW = _B // _NW


@functools.cache
def _sc_pairs():
    return pl.kernel(
        _sc_pairs_body,
        out_type=(jax.ShapeDtypeStruct((_NW, _PPW, _TW), _F32),
                  jax.ShapeDtypeStruct((_NW, _PPW, _TW), _F32)),
        mesh=_mesh(),
        compiler_params=pltpu.CompilerParams(needs_layout_passes=False,
                                             use_tc_tiling_on_sc=False),
        scratch_types=[
            pltpu.VMEM((_PPW,), _I32),
            pltpu.VMEM((_PPW,), _I32),
            pltpu.VMEM((_PPW, _TW), _F32),
            pltpu.VMEM((_PPW, _TW), _F32),
            pltpu.SemaphoreType.DMA,
            pltpu.SemaphoreType.DMA,
        ],
    )


def kernel(users, items, adj_list, adj_relation, entity_emb, user_emb,
           We1, be1, We2, be2, Wd1, bd1, Wd2, bd2,
           Wr1, lw1, lb1, Wr2, lw2, lb2):
    pad = _NP - _N
    emb = jnp.concatenate([entity_emb, user_emb], axis=0)
    emb_p = jnp.pad(emb, ((0, pad), (0, 0)))
    adj_p = jnp.pad(adj_list.astype(_I32), ((0, pad), (0, 0)))
    rel_p = jnp.pad(adj_relation.astype(_I32), ((0, pad), (0, 0)))

    t1, pg1, qc1, enc, sq = pl.pallas_call(
        _tc_encode_body,
        grid=(_NBLK,),
        in_specs=[
            _rows(64), _rows(_K),
            _full((64, 128)), _full((1, 128)), _full((128, 32)),
            _full((1, 32)), _full((32, 128)), _full((1, 128)),
            _full((128, 64)), _full((1, 64)), _full((_R, 64, 32)),
        ],
        out_specs=[
            _rows(_TW), _rows(_K), _rows(_K), _rows(32),
            pl.BlockSpec((1, 1), lambda i: (0, 0),
                         memory_space=pltpu.SMEM),
        ],
        out_shape=[
            jax.ShapeDtypeStruct((_NP, _TW), _F32),
            jax.ShapeDtypeStruct((_NP, _K), _F32),
            jax.ShapeDtypeStruct((_NP, _K), _I32),
            jax.ShapeDtypeStruct((_NP, 32), _F32),
            jax.ShapeDtypeStruct((1, 1), _F32),
        ],
    )(emb_p, rel_p, We1, be1.reshape(1, -1), We2, be2.reshape(1, -1),
      Wd1, bd1.reshape(1, -1), Wd2, bd2.reshape(1, -1), Wr1)
    ae_loss = sq[0, 0] / (_N * 64)

    adj_r = adj_p.reshape(_NW, _CH, _G * _K)

    agg1 = _sc_agg()(t1, adj_r, qc1.reshape(_NW, _CPW, _K),
                     pg1.reshape(_NW, _CPW, _K)).reshape(_NP, 32)

    x1, t2, pg2, qc2 = pl.pallas_call(
        _tc_combine_body,
        grid=(_NBLK,),
        in_specs=[
            _rows(32), _rows(32), _rows(_K),
            _full((64, 32)), _full((1, 32)), _full((_R, 64, 32)),
        ],
        out_specs=[_rows(32), _rows(_TW), _rows(_K), _rows(_K)],
        out_shape=[
            jax.ShapeDtypeStruct((_NP, 32), _F32),
            jax.ShapeDtypeStruct((_NP, _TW), _F32),
            jax.ShapeDtypeStruct((_NP, _K), _F32),
            jax.ShapeDtypeStruct((_NP, _K), _I32),
        ],
    )(enc, agg1, rel_p, lw1, lb1.reshape(1, -1), Wr2)

    agg2 = _sc_agg()(t2, adj_r, qc2.reshape(_NW, _CPW, _K),
                     pg2.reshape(_NW, _CPW, _K)).reshape(_NP, 32)

    fin = pl.pallas_call(
        _tc_final_body,
        grid=(_NBLK,),
        in_specs=[_rows(32), _rows(32), _full((64, 16)), _full((1, 16))],
        out_specs=[_rows(_TW)],
        out_shape=[jax.ShapeDtypeStruct((_NP, _TW), _F32)],
    )(x1, agg2, lw2, lb2.reshape(1, -1))[0]

    uidx = (users.astype(_I32) + _N_ENT).reshape(_NW, _PPW)
    iidx = items.astype(_I32).reshape(_NW, _PPW)
    urows, irows = _sc_pairs()(fin, uidx, iidx)

    scores = pl.pallas_call(
        _tc_score_body,
        grid=(_B // _BLK,),
        in_specs=[_rows(_TW), _rows(_TW)],
        out_specs=[pl.BlockSpec((_BLK, 1), lambda i: (i, 0))],
        out_shape=[jax.ShapeDtypeStruct((_B, 1), _F32)],
    )(urows.reshape(_B, _TW), irows.reshape(_B, _TW))[0]

    return scores.reshape(_B), ae_loss


# speedup vs baseline: 12.9304x; 1.3531x over previous
import functools

import jax
import jax.numpy as jnp
from jax import lax
from jax.experimental import pallas as pl
from jax.experimental.pallas import tpu as pltpu
from jax.experimental.pallas import tpu_sc as plsc

_N_ENT = 45000
_N = 50000
_K = 16
_R = 4
_B = 4096
_NW = 32
_CPW = 1568
_NP = _NW * _CPW
_G = 8
_CH = _CPW // _G
_NBUF = 4
_TW = 48
_BLK = 3584
_NBLK = _NP // _BLK

_F32 = jnp.float32
_I32 = jnp.int32
_PREC = lax.Precision.DEFAULT


def _leaky(x):
    return jnp.where(x >= 0, x, 0.01 * x)


def _gate_precompute(x, rel, wr):
    wsum = jnp.sum(wr, axis=-1)
    p = lax.dot_general(x, wsum[:, :32], (((1,), (1,)), ((), ())),
                        precision=_PREC, preferred_element_type=_F32)
    q = lax.dot_general(x, wsum[:, 32:], (((1,), (1,)), ((), ())),
                        precision=_PREC, preferred_element_type=_F32)
    pg = jnp.zeros(rel.shape, _F32)
    for r in range(_R):
        pg = pg + jnp.where(rel == r, p[:, r][:, None], 0.0)
    return pg, q


def _tc_encode_body(emb_ref, rel_ref, we1, be1, we2, be2, wd1, bd1, wd2, bd2,
                    wr1, t1_ref, pg_ref, qc_ref, enc_ref, sq_ref):
    i = pl.program_id(0)
    emb = emb_ref[...]
    h = jnp.maximum(jnp.dot(emb, we1[...], precision=_PREC,
                            preferred_element_type=_F32) + be1[...], 0.0)
    enc = jnp.dot(h, we2[...], precision=_PREC,
                  preferred_element_type=_F32) + be2[...]
    enc_ref[...] = enc
    h2 = jnp.maximum(jnp.dot(enc, wd1[...], precision=_PREC,
                             preferred_element_type=_F32) + bd1[...], 0.0)
    dec = jnp.dot(h2, wd2[...], precision=_PREC,
                  preferred_element_type=_F32) + bd2[...]
    d = dec - emb
    rows = i * _BLK + lax.broadcasted_iota(_I32, (_BLK, 1), 0)
    d = d * (rows < _N).astype(_F32)
    blksum = jnp.sum(d * d)

    @pl.when(i == 0)
    def _():
        sq_ref[0, 0] = 0.0

    sq_ref[0, 0] += blksum
    rel = rel_ref[...]
    pg, q = _gate_precompute(enc, rel, wr1[...])
    pg_ref[...] = pg
    qc_ref[...] = rel + 32
    t1_ref[...] = jnp.concatenate(
        [enc, q, jnp.zeros((_BLK, _TW - 36), _F32)], axis=1)


def _tc_combine_body(prev_ref, agg_ref, rel_ref, lw, lb, wr2,
                     x_ref, t2_ref, pg_ref, qc_ref):
    prev = prev_ref[...]
    agg = agg_ref[...]
    w = lw[...]
    comb = (jnp.dot(prev, w[:32, :], precision=_PREC,
                    preferred_element_type=_F32)
            + jnp.dot(agg, w[32:, :], precision=_PREC,
                      preferred_element_type=_F32) + lb[...])
    x = _leaky(comb)
    x_ref[...] = x
    rel = rel_ref[...]
    pg, q = _gate_precompute(x, rel, wr2[...])
    pg_ref[...] = pg
    qc_ref[...] = rel + 32
    t2_ref[...] = jnp.concatenate(
        [x, q, jnp.zeros((_BLK, _TW - 36), _F32)], axis=1)


def _tc_final_body(x1_ref, agg_ref, lw, lb, fin_ref):
    x1 = x1_ref[...]
    w = lw[...]
    comb = (jnp.dot(x1, w[:32, :], precision=_PREC,
                    preferred_element_type=_F32)
            + jnp.dot(agg_ref[...], w[32:, :], precision=_PREC,
                      preferred_element_type=_F32) + lb[...])
    fin_ref[...] = jnp.concatenate([x1, _leaky(comb)], axis=1)


_SBLK = 1024


def _tc_score_body(u_ref, v_ref, o_ref):
    o_ref[...] = jnp.sum(u_ref[...] * v_ref[...], axis=1, keepdims=True)


def _full(shape):
    return pl.BlockSpec(shape, lambda i: tuple(0 for _ in shape))


def _rows(width, dtype=_F32):
    del dtype
    return pl.BlockSpec((_BLK, width), lambda i: (i, 0))


@functools.cache
def _mesh():
    return plsc.VectorSubcoreMesh(core_axis_name="c", subcore_axis_name="s")


def _wid():
    return lax.axis_index("s") * 2 + lax.axis_index("c")


def _sc_agg_body(t_hbm, adj_hbm, qc_hbm, pg_hbm, out_hbm,
                 adj_v, qc_v, pg_v, rows_v, out_v,
                 gs0, gs1, gs2, gs3, os0, os1, os2, os3):
    gsems = (gs0, gs1, gs2, gs3)
    osems = (os0, os1, os2, os3)
    w = _wid()
    pltpu.sync_copy(adj_hbm.at[w], adj_v)
    pltpu.sync_copy(qc_hbm.at[w], qc_v)
    pltpu.sync_copy(pg_hbm.at[w], pg_v)
    for b in range(_NBUF):
        pltpu.async_copy(t_hbm.at[adj_v.at[b]], rows_v.at[b], gsems[b])

    def chunk(c, b):
        pltpu.make_async_copy(t_hbm.at[adj_v.at[c]], rows_v.at[b],
                              gsems[b]).wait()
        @pl.when(c >= _NBUF)
        def _():
            pltpu.make_async_copy(
                out_v.at[b], out_hbm.at[w, pl.ds((c - _NBUF) * _G, _G)],
                osems[b]).wait()

        for g in range(_G):
            node = c * _G + g
            rowi = g * _K + lax.iota(_I32, 16)
            qs = plsc.load_gather(rows_v.at[b], [rowi, qc_v[node, :]])
            s = pg_v[node, :] + qs
            gate = (1.0 / _K) / (1.0 + jnp.exp(-s))
            z = jnp.zeros((16,), _F32)
            acc = [z, z, z, z]
            for k in range(_K):
                gk = gate[k]
                acc[k % 2] = acc[k % 2] + gk * rows_v[b, g * _K + k, 0:16]
                acc[2 + k % 2] = (acc[2 + k % 2]
                                  + gk * rows_v[b, g * _K + k, 16:32])
            out_v[b, g, 0:16] = acc[0] + acc[1]
            out_v[b, g, 16:32] = acc[2] + acc[3]
        pltpu.async_copy(out_v.at[b], out_hbm.at[w, pl.ds(c * _G, _G)],
                         osems[b])

        @pl.when(c + _NBUF < _CH)
        def _():
            pltpu.async_copy(t_hbm.at[adj_v.at[c + _NBUF]], rows_v.at[b],
                             gsems[b])

    def body(i, carry):
        c0 = i * _NBUF
        for b in range(_NBUF):
            chunk(c0 + b, b)
        return carry

    lax.fori_loop(0, _CH // _NBUF, body, 0)
    for b in range(_NBUF):
        c_last = _CH - _NBUF + b
        pltpu.make_async_copy(
            out_v.at[b], out_hbm.at[w, pl.ds(c_last * _G, _G)],
            osems[b]).wait()


@functools.cache
def _sc_agg():
    return pl.kernel(
        _sc_agg_body,
        out_type=jax.ShapeDtypeStruct((_NW, _CPW, 32), _F32),
        mesh=_mesh(),
        compiler_params=pltpu.CompilerParams(needs_layout_passes=False,
                                             use_tc_tiling_on_sc=False),
        scratch_types=[
            pltpu.VMEM((_CH, _G * _K), _I32),
            pltpu.VMEM((_CPW, _K), _I32),
            pltpu.VMEM((_CPW, _K), _F32),
            pltpu.VMEM((_NBUF, _G * _K, _TW), _F32),
            pltpu.VMEM((_NBUF, _G, 32), _F32),
        ] + [pltpu.SemaphoreType.DMA] * (2 * _NBUF),
    )


def _sc_pairs_body(fin_hbm, ui_hbm, ii_hbm, uo_hbm, io_hbm,
                   ui_v, ii_v, ur_v, ir_v, s1, s2):
    w = _wid()
    pltpu.sync_copy(ui_hbm.at[w], ui_v)
    pltpu.sync_copy(ii_hbm.at[w], ii_v)
    cu = pltpu.async_copy(fin_hbm.at[ui_v], ur_v, s1)
    ci = pltpu.async_copy(fin_hbm.at[ii_v], ir_v, s2)
    cu.wait()
    ci.wait()
    pltpu.sync_copy(ur_v, uo_hbm.at[w])
    pltpu.sync_copy(ir_v, io_hbm.at[w])


_PPW = _B // _NW


@functools.cache
def _sc_pairs():
    return pl.kernel(
        _sc_pairs_body,
        out_type=(jax.ShapeDtypeStruct((_NW, _PPW, _TW), _F32),
                  jax.ShapeDtypeStruct((_NW, _PPW, _TW), _F32)),
        mesh=_mesh(),
        compiler_params=pltpu.CompilerParams(needs_layout_passes=False,
                                             use_tc_tiling_on_sc=False),
        scratch_types=[
            pltpu.VMEM((_PPW,), _I32),
            pltpu.VMEM((_PPW,), _I32),
            pltpu.VMEM((_PPW, _TW), _F32),
            pltpu.VMEM((_PPW, _TW), _F32),
            pltpu.SemaphoreType.DMA,
            pltpu.SemaphoreType.DMA,
        ],
    )


def kernel(users, items, adj_list, adj_relation, entity_emb, user_emb,
           We1, be1, We2, be2, Wd1, bd1, Wd2, bd2,
           Wr1, lw1, lb1, Wr2, lw2, lb2):
    pad = _NP - _N
    emb = jnp.concatenate([entity_emb, user_emb], axis=0)
    emb_p = jnp.pad(emb, ((0, pad), (0, 0)))
    adj_p = jnp.pad(adj_list.astype(_I32), ((0, pad), (0, 0)))
    rel_p = jnp.pad(adj_relation.astype(_I32), ((0, pad), (0, 0)))

    t1, pg1, qc1, enc, sq = pl.pallas_call(
        _tc_encode_body,
        grid=(_NBLK,),
        in_specs=[
            _rows(64), _rows(_K),
            _full((64, 128)), _full((1, 128)), _full((128, 32)),
            _full((1, 32)), _full((32, 128)), _full((1, 128)),
            _full((128, 64)), _full((1, 64)), _full((_R, 64, 32)),
        ],
        out_specs=[
            _rows(_TW), _rows(_K), _rows(_K), _rows(32),
            pl.BlockSpec((1, 1), lambda i: (0, 0),
                         memory_space=pltpu.SMEM),
        ],
        out_shape=[
            jax.ShapeDtypeStruct((_NP, _TW), _F32),
            jax.ShapeDtypeStruct((_NP, _K), _F32),
            jax.ShapeDtypeStruct((_NP, _K), _I32),
            jax.ShapeDtypeStruct((_NP, 32), _F32),
            jax.ShapeDtypeStruct((1, 1), _F32),
        ],
    )(emb_p, rel_p, We1, be1.reshape(1, -1), We2, be2.reshape(1, -1),
      Wd1, bd1.reshape(1, -1), Wd2, bd2.reshape(1, -1), Wr1)
    ae_loss = sq[0, 0] / (_N * 64)

    adj_r = adj_p.reshape(_NW, _CH, _G * _K)

    agg1 = _sc_agg()(t1, adj_r, qc1.reshape(_NW, _CPW, _K),
                     pg1.reshape(_NW, _CPW, _K)).reshape(_NP, 32)

    x1, t2, pg2, qc2 = pl.pallas_call(
        _tc_combine_body,
        grid=(_NBLK,),
        in_specs=[
            _rows(32), _rows(32), _rows(_K),
            _full((64, 32)), _full((1, 32)), _full((_R, 64, 32)),
        ],
        out_specs=[_rows(32), _rows(_TW), _rows(_K), _rows(_K)],
        out_shape=[
            jax.ShapeDtypeStruct((_NP, 32), _F32),
            jax.ShapeDtypeStruct((_NP, _TW), _F32),
            jax.ShapeDtypeStruct((_NP, _K), _F32),
            jax.ShapeDtypeStruct((_NP, _K), _I32),
        ],
    )(enc, agg1, rel_p, lw1, lb1.reshape(1, -1), Wr2)

    agg2 = _sc_agg()(t2, adj_r, qc2.reshape(_NW, _CPW, _K),
                     pg2.reshape(_NW, _CPW, _K)).reshape(_NP, 32)

    fin = pl.pallas_call(
        _tc_final_body,
        grid=(_NBLK,),
        in_specs=[_rows(32), _rows(32), _full((64, 16)), _full((1, 16))],
        out_specs=[_rows(_TW)],
        out_shape=[jax.ShapeDtypeStruct((_NP, _TW), _F32)],
    )(x1, agg2, lw2, lb2.reshape(1, -1))[0]

    uidx = (users.astype(_I32) + _N_ENT).reshape(_NW, _PPW)
    iidx = items.astype(_I32).reshape(_NW, _PPW)
    urows, irows = _sc_pairs()(fin, uidx, iidx)

    scores = pl.pallas_call(
        _tc_score_body,
        grid=(_B // _SBLK,),
        in_specs=[pl.BlockSpec((_SBLK, _TW), lambda i: (i, 0)),
                  pl.BlockSpec((_SBLK, _TW), lambda i: (i, 0))],
        out_specs=[pl.BlockSpec((_SBLK, 1), lambda i: (i, 0))],
        out_shape=[jax.ShapeDtypeStruct((_B, 1), _F32)],
    )(urows.reshape(_B, _TW), irows.reshape(_B, _TW))[0]

    return scores.reshape(_B), ae_loss


# speedup vs baseline: 12.9478x; 1.0013x over previous
import functools

import jax
import jax.numpy as jnp
from jax import lax
from jax.experimental import pallas as pl
from jax.experimental.pallas import tpu as pltpu
from jax.experimental.pallas import tpu_sc as plsc

_N_ENT = 45000
_N = 50000
_K = 16
_R = 4
_B = 4096
_NW = 32
_CPW = 1568
_NP = _NW * _CPW
_G = 8
_CH = _CPW // _G
_NBUF = 4
_TW = 48
_BLK = 3584
_NBLK = _NP // _BLK

_F32 = jnp.float32
_I32 = jnp.int32
_PREC = lax.Precision.DEFAULT


def _leaky(x):
    return jnp.where(x >= 0, x, 0.01 * x)


def _gate_precompute(x, rel, wr):
    wsum = jnp.sum(wr, axis=-1)
    p = lax.dot_general(x, wsum[:, :32], (((1,), (1,)), ((), ())),
                        precision=_PREC, preferred_element_type=_F32)
    q = lax.dot_general(x, wsum[:, 32:], (((1,), (1,)), ((), ())),
                        precision=_PREC, preferred_element_type=_F32)
    pg = jnp.zeros(rel.shape, _F32)
    for r in range(_R):
        pg = pg + jnp.where(rel == r, p[:, r][:, None], 0.0)
    return pg, q


def _tc_encode_body(emb_ref, rel_ref, we1, be1, we2, be2, wd1, bd1, wd2, bd2,
                    wr1, tx_ref, tq_ref, pg_ref, enc_ref, sq_ref):
    i = pl.program_id(0)
    emb = emb_ref[...]
    h = jnp.maximum(jnp.dot(emb, we1[...], precision=_PREC,
                            preferred_element_type=_F32) + be1[...], 0.0)
    enc = jnp.dot(h, we2[...], precision=_PREC,
                  preferred_element_type=_F32) + be2[...]
    enc_ref[...] = enc
    h2 = jnp.maximum(jnp.dot(enc, wd1[...], precision=_PREC,
                             preferred_element_type=_F32) + bd1[...], 0.0)
    dec = jnp.dot(h2, wd2[...], precision=_PREC,
                  preferred_element_type=_F32) + bd2[...]
    d = dec - emb
    rows = i * _BLK + lax.broadcasted_iota(_I32, (_BLK, 1), 0)
    d = d * (rows < _N).astype(_F32)
    blksum = jnp.sum(d * d)

    @pl.when(i == 0)
    def _():
        sq_ref[0, 0] = 0.0

    sq_ref[0, 0] += blksum
    rel = rel_ref[...]
    pg, q = _gate_precompute(enc, rel, wr1[...])
    pg_ref[...] = pg
    tx_ref[...] = enc.astype(jnp.bfloat16)
    tq_ref[...] = q


def _tc_combine_body(prev_ref, agg_ref, rel_ref, lw, lb, wr2,
                     x_ref, tx_ref, tq_ref, pg_ref):
    prev = prev_ref[...]
    agg = agg_ref[...]
    w = lw[...]
    comb = (jnp.dot(prev, w[:32, :], precision=_PREC,
                    preferred_element_type=_F32)
            + jnp.dot(agg, w[32:, :], precision=_PREC,
                      preferred_element_type=_F32) + lb[...])
    x = _leaky(comb)
    x_ref[...] = x
    rel = rel_ref[...]
    pg, q = _gate_precompute(x, rel, wr2[...])
    pg_ref[...] = pg
    tx_ref[...] = x.astype(jnp.bfloat16)
    tq_ref[...] = q


def _tc_final_body(x1_ref, agg_ref, lw, lb, fin_ref):
    x1 = x1_ref[...]
    w = lw[...]
    comb = (jnp.dot(x1, w[:32, :], precision=_PREC,
                    preferred_element_type=_F32)
            + jnp.dot(agg_ref[...], w[32:, :], precision=_PREC,
                      preferred_element_type=_F32) + lb[...])
    fin_ref[...] = jnp.concatenate([x1, _leaky(comb)], axis=1)


_SBLK = 1024


def _tc_score_body(u_ref, v_ref, o_ref):
    o_ref[...] = jnp.sum(u_ref[...] * v_ref[...], axis=1, keepdims=True)


def _full(shape):
    return pl.BlockSpec(shape, lambda i: tuple(0 for _ in shape))


def _rows(width, dtype=_F32):
    del dtype
    return pl.BlockSpec((_BLK, width), lambda i: (i, 0))


@functools.cache
def _mesh():
    return plsc.VectorSubcoreMesh(core_axis_name="c", subcore_axis_name="s")


def _wid():
    return lax.axis_index("s") * 2 + lax.axis_index("c")


def _sc_agg_body(tx_hbm, tq_hbm, adj_hbm, rel_hbm, pg_hbm, out_hbm,
                 adj_v, rel_v, pg_v, rowsx_v, rowsq_v, out_v,
                 gx0, gx1, gx2, gx3, gq0, gq1, gq2, gq3,
                 os0, os1, os2, os3):
    gxsems = (gx0, gx1, gx2, gx3)
    gqsems = (gq0, gq1, gq2, gq3)
    osems = (os0, os1, os2, os3)
    w = _wid()
    pltpu.sync_copy(adj_hbm.at[w], adj_v)
    pltpu.sync_copy(rel_hbm.at[w], rel_v)
    pltpu.sync_copy(pg_hbm.at[w], pg_v)
    for b in range(_NBUF):
        pltpu.async_copy(tx_hbm.at[adj_v.at[b]], rowsx_v.at[b], gxsems[b])
        pltpu.async_copy(tq_hbm.at[adj_v.at[b]], rowsq_v.at[b], gqsems[b])

    def chunk(c, b):
        pltpu.make_async_copy(tx_hbm.at[adj_v.at[c]], rowsx_v.at[b],
                              gxsems[b]).wait()
        pltpu.make_async_copy(tq_hbm.at[adj_v.at[c]], rowsq_v.at[b],
                              gqsems[b]).wait()
        @pl.when(c >= _NBUF)
        def _():
            pltpu.make_async_copy(
                out_v.at[b], out_hbm.at[w, pl.ds((c - _NBUF) * _G, _G)],
                osems[b]).wait()

        for g in range(_G):
            node = c * _G + g
            rowi = g * _K + lax.iota(_I32, 16)
            qs = plsc.load_gather(rowsq_v.at[b], [rowi, rel_v[node, :]])
            s = pg_v[node, :] + qs
            gate = (1.0 / _K) / (1.0 + jnp.exp(-s))
            z = jnp.zeros((16,), _F32)
            acc = [z, z, z, z]
            for k in range(_K):
                gk = gate[k]
                xa, xb = plsc.unpack(rowsx_v[b, g * _K + k, :],
                                     format=plsc.PackFormat.INTERLEAVED)
                acc[k % 2] = acc[k % 2] + gk * xa
                acc[2 + k % 2] = acc[2 + k % 2] + gk * xb
            out_v[b, g, 0:16] = acc[0] + acc[1]
            out_v[b, g, 16:32] = acc[2] + acc[3]
        pltpu.async_copy(out_v.at[b], out_hbm.at[w, pl.ds(c * _G, _G)],
                         osems[b])

        @pl.when(c + _NBUF < _CH)
        def _():
            pltpu.async_copy(tx_hbm.at[adj_v.at[c + _NBUF]], rowsx_v.at[b],
                             gxsems[b])
            pltpu.async_copy(tq_hbm.at[adj_v.at[c + _NBUF]], rowsq_v.at[b],
                             gqsems[b])

    def body(i, carry):
        c0 = i * _NBUF
        for b in range(_NBUF):
            chunk(c0 + b, b)
        return carry

    lax.fori_loop(0, _CH // _NBUF, body, 0)
    for b in range(_NBUF):
        c_last = _CH - _NBUF + b
        pltpu.make_async_copy(
            out_v.at[b], out_hbm.at[w, pl.ds(c_last * _G, _G)],
            osems[b]).wait()


@functools.cache
def _sc_agg():
    return pl.kernel(
        _sc_agg_body,
        out_type=jax.ShapeDtypeStruct((_NW, _CPW, 32), _F32),
        mesh=_mesh(),
        compiler_params=pltpu.CompilerParams(needs_layout_passes=False,
                                             use_tc_tiling_on_sc=False),
        scratch_types=[
            pltpu.VMEM((_CH, _G * _K), _I32),
            pltpu.VMEM((_CPW, _K), _I32),
            pltpu.VMEM((_CPW, _K), _F32),
            pltpu.VMEM((_NBUF, _G * _K, 32), jnp.bfloat16),
            pltpu.VMEM((_NBUF, _G * _K, _R), _F32),
            pltpu.VMEM((_NBUF, _G, 32), _F32),
        ] + [pltpu.SemaphoreType.DMA] * (3 * _NBUF),
    )


def _sc_pairs_body(fin_hbm, ui_hbm, ii_hbm, uo_hbm, io_hbm,
                   ui_v, ii_v, ur_v, ir_v, s1, s2):
    w = _wid()
    pltpu.sync_copy(ui_hbm.at[w], ui_v)
    pltpu.sync_copy(ii_hbm.at[w], ii_v)
    cu = pltpu.async_copy(fin_hbm.at[ui_v], ur_v, s1)
    ci = pltpu.async_copy(fin_hbm.at[ii_v], ir_v, s2)
    cu.wait()
    ci.wait()
    pltpu.sync_copy(ur_v, uo_hbm.at[w])
    pltpu.sync_copy(ir_v, io_hbm.at[w])


_PPW = _B // _NW


@functools.cache
def _sc_pairs():
    return pl.kernel(
        _sc_pairs_body,
        out_type=(jax.ShapeDtypeStruct((_NW, _PPW, _TW), _F32),
                  jax.ShapeDtypeStruct((_NW, _PPW, _TW), _F32)),
        mesh=_mesh(),
        compiler_params=pltpu.CompilerParams(needs_layout_passes=False,
                                             use_tc_tiling_on_sc=False),
        scratch_types=[
            pltpu.VMEM((_PPW,), _I32),
            pltpu.VMEM((_PPW,), _I32),
            pltpu.VMEM((_PPW, _TW), _F32),
            pltpu.VMEM((_PPW, _TW), _F32),
            pltpu.SemaphoreType.DMA,
            pltpu.SemaphoreType.DMA,
        ],
    )


def kernel(users, items, adj_list, adj_relation, entity_emb, user_emb,
           We1, be1, We2, be2, Wd1, bd1, Wd2, bd2,
           Wr1, lw1, lb1, Wr2, lw2, lb2):
    pad = _NP - _N
    emb = jnp.concatenate([entity_emb, user_emb], axis=0)
    emb_p = jnp.pad(emb, ((0, pad), (0, 0)))
    adj_p = jnp.pad(adj_list.astype(_I32), ((0, pad), (0, 0)))
    rel_p = jnp.pad(adj_relation.astype(_I32), ((0, pad), (0, 0)))

    perm = jnp.arange(32).reshape(16, 2).T.reshape(32)
    lw1_eff = jnp.concatenate([lw1[:32], lw1[32:][perm]], axis=0)
    lw2_eff = jnp.concatenate([lw2[:32], lw2[32:][perm]], axis=0)

    tx1, tq1, pg1, enc, sq = pl.pallas_call(
        _tc_encode_body,
        grid=(_NBLK,),
        in_specs=[
            _rows(64), _rows(_K),
            _full((64, 128)), _full((1, 128)), _full((128, 32)),
            _full((1, 32)), _full((32, 128)), _full((1, 128)),
            _full((128, 64)), _full((1, 64)), _full((_R, 64, 32)),
        ],
        out_specs=[
            _rows(32), _rows(_R), _rows(_K), _rows(32),
            pl.BlockSpec((1, 1), lambda i: (0, 0),
                         memory_space=pltpu.SMEM),
        ],
        out_shape=[
            jax.ShapeDtypeStruct((_NP, 32), jnp.bfloat16),
            jax.ShapeDtypeStruct((_NP, _R), _F32),
            jax.ShapeDtypeStruct((_NP, _K), _F32),
            jax.ShapeDtypeStruct((_NP, 32), _F32),
            jax.ShapeDtypeStruct((1, 1), _F32),
        ],
    )(emb_p, rel_p, We1, be1.reshape(1, -1), We2, be2.reshape(1, -1),
      Wd1, bd1.reshape(1, -1), Wd2, bd2.reshape(1, -1), Wr1)
    ae_loss = sq[0, 0] / (_N * 64)

    adj_r = adj_p.reshape(_NW, _CH, _G * _K)
    rel_r = rel_p.reshape(_NW, _CPW, _K)

    agg1 = _sc_agg()(tx1, tq1, adj_r, rel_r,
                     pg1.reshape(_NW, _CPW, _K)).reshape(_NP, 32)

    x1, tx2, tq2, pg2 = pl.pallas_call(
        _tc_combine_body,
        grid=(_NBLK,),
        in_specs=[
            _rows(32), _rows(32), _rows(_K),
            _full((64, 32)), _full((1, 32)), _full((_R, 64, 32)),
        ],
        out_specs=[_rows(32), _rows(32), _rows(_R), _rows(_K)],
        out_shape=[
            jax.ShapeDtypeStruct((_NP, 32), _F32),
            jax.ShapeDtypeStruct((_NP, 32), jnp.bfloat16),
            jax.ShapeDtypeStruct((_NP, _R), _F32),
            jax.ShapeDtypeStruct((_NP, _K), _F32),
        ],
    )(enc, agg1, rel_p, lw1_eff, lb1.reshape(1, -1), Wr2)

    agg2 = _sc_agg()(tx2, tq2, adj_r, rel_r,
                     pg2.reshape(_NW, _CPW, _K)).reshape(_NP, 32)

    fin = pl.pallas_call(
        _tc_final_body,
        grid=(_NBLK,),
        in_specs=[_rows(32), _rows(32), _full((64, 16)), _full((1, 16))],
        out_specs=[_rows(_TW)],
        out_shape=[jax.ShapeDtypeStruct((_NP, _TW), _F32)],
    )(x1, agg2, lw2_eff, lb2.reshape(1, -1))[0]

    uidx = (users.astype(_I32) + _N_ENT).reshape(_NW, _PPW)
    iidx = items.astype(_I32).reshape(_NW, _PPW)
    urows, irows = _sc_pairs()(fin, uidx, iidx)

    scores = pl.pallas_call(
        _tc_score_body,
        grid=(_B // _SBLK,),
        in_specs=[pl.BlockSpec((_SBLK, _TW), lambda i: (i, 0)),
                  pl.BlockSpec((_SBLK, _TW), lambda i: (i, 0))],
        out_specs=[pl.BlockSpec((_SBLK, 1), lambda i: (i, 0))],
        out_shape=[jax.ShapeDtypeStruct((_B, 1), _F32)],
    )(urows.reshape(_B, _TW), irows.reshape(_B, _TW))[0]

    return scores.reshape(_B), ae_loss


# speedup vs baseline: 14.3669x; 1.1096x over previous
import functools

import jax
import jax.numpy as jnp
from jax import lax
from jax.experimental import pallas as pl
from jax.experimental.pallas import tpu as pltpu
from jax.experimental.pallas import tpu_sc as plsc

_N_ENT = 45000
_N = 50000
_K = 16
_R = 4
_B = 4096
_NW = 32
_CPW = 1568
_NP = _NW * _CPW
_G = 8
_CH = _CPW // _G
_NBUF = 4
_TW = 48
_BLK = 3584
_NBLK = _NP // _BLK

_F32 = jnp.float32
_I32 = jnp.int32
_PREC = lax.Precision.DEFAULT


def _leaky(x):
    return jnp.where(x >= 0, x, 0.01 * x)


def _gate_precompute(x, rel, wr):
    wsum = jnp.sum(wr, axis=-1)
    p = lax.dot_general(x, wsum[:, :32], (((1,), (1,)), ((), ())),
                        precision=_PREC, preferred_element_type=_F32)
    q = lax.dot_general(x, wsum[:, 32:], (((1,), (1,)), ((), ())),
                        precision=_PREC, preferred_element_type=_F32)
    pg = jnp.zeros(rel.shape, _F32)
    for r in range(_R):
        pg = pg + jnp.where(rel == r, p[:, r][:, None], 0.0)
    return pg, q


def _tc_encode_body(emb_ref, rel_ref, we1, be1, we2, be2, wd1, bd1, wd2, bd2,
                    wr1, tx_ref, tq_ref, pg_ref, enc_ref, sq_ref):
    i = pl.program_id(0)
    emb = emb_ref[...]
    h = jnp.maximum(jnp.dot(emb, we1[...], precision=_PREC,
                            preferred_element_type=_F32) + be1[...], 0.0)
    enc = jnp.dot(h, we2[...], precision=_PREC,
                  preferred_element_type=_F32) + be2[...]
    enc_ref[...] = enc
    h2 = jnp.maximum(jnp.dot(enc, wd1[...], precision=_PREC,
                             preferred_element_type=_F32) + bd1[...], 0.0)
    dec = jnp.dot(h2, wd2[...], precision=_PREC,
                  preferred_element_type=_F32) + bd2[...]
    d = dec - emb
    rows = i * _BLK + lax.broadcasted_iota(_I32, (_BLK, 1), 0)
    d = d * (rows < _N).astype(_F32)
    blksum = jnp.sum(d * d)

    @pl.when(i == 0)
    def _():
        sq_ref[0, 0] = 0.0

    sq_ref[0, 0] += blksum
    rel = rel_ref[...]
    pg, q = _gate_precompute(enc, rel, wr1[...])
    pg_ref[...] = pg
    tx_ref[...] = enc
    tq_ref[...] = q


def _tc_combine_body(prev_ref, agg_ref, rel_ref, lw, lb, wr2,
                     x_ref, tx_ref, tq_ref, pg_ref):
    prev = prev_ref[...]
    agg = agg_ref[...]
    w = lw[...]
    comb = (jnp.dot(prev, w[:32, :], precision=_PREC,
                    preferred_element_type=_F32)
            + jnp.dot(agg, w[32:, :], precision=_PREC,
                      preferred_element_type=_F32) + lb[...])
    x = _leaky(comb)
    x_ref[...] = x
    rel = rel_ref[...]
    pg, q = _gate_precompute(x, rel, wr2[...])
    pg_ref[...] = pg
    tx_ref[...] = x
    tq_ref[...] = q


def _tc_final_body(x1_ref, agg_ref, lw, lb, fin_ref):
    x1 = x1_ref[...]
    w = lw[...]
    comb = (jnp.dot(x1, w[:32, :], precision=_PREC,
                    preferred_element_type=_F32)
            + jnp.dot(agg_ref[...], w[32:, :], precision=_PREC,
                      preferred_element_type=_F32) + lb[...])
    fin_ref[...] = jnp.concatenate([x1, _leaky(comb)], axis=1)


_SBLK = 1024


def _tc_score_body(u_ref, v_ref, o_ref):
    o_ref[...] = jnp.sum(u_ref[...] * v_ref[...], axis=1, keepdims=True)


def _full(shape):
    return pl.BlockSpec(shape, lambda i: tuple(0 for _ in shape))


def _rows(width, dtype=_F32):
    del dtype
    return pl.BlockSpec((_BLK, width), lambda i: (i, 0))


@functools.cache
def _mesh():
    return plsc.VectorSubcoreMesh(core_axis_name="c", subcore_axis_name="s")


def _wid():
    return lax.axis_index("s") * 2 + lax.axis_index("c")


def _sc_agg_body(tx_hbm, tq_hbm, adj_hbm, rel_hbm, pg_hbm, out_hbm,
                 adj_v, rel_v, pg_v, rowsx_v, rowsq_v, out_v,
                 gx0, gx1, gx2, gx3, gq0, gq1, gq2, gq3,
                 os0, os1, os2, os3):
    gxsems = (gx0, gx1, gx2, gx3)
    gqsems = (gq0, gq1, gq2, gq3)
    osems = (os0, os1, os2, os3)
    w = _wid()
    pltpu.sync_copy(adj_hbm.at[w], adj_v)
    pltpu.sync_copy(rel_hbm.at[w], rel_v)
    pltpu.sync_copy(pg_hbm.at[w], pg_v)
    for b in range(_NBUF):
        pltpu.async_copy(tx_hbm.at[adj_v.at[b]], rowsx_v.at[b], gxsems[b])
        pltpu.async_copy(tq_hbm.at[adj_v.at[b]], rowsq_v.at[b], gqsems[b])

    def chunk(c, b):
        pltpu.make_async_copy(tx_hbm.at[adj_v.at[c]], rowsx_v.at[b],
                              gxsems[b]).wait()
        pltpu.make_async_copy(tq_hbm.at[adj_v.at[c]], rowsq_v.at[b],
                              gqsems[b]).wait()
        @pl.when(c >= _NBUF)
        def _():
            pltpu.make_async_copy(
                out_v.at[b], out_hbm.at[w, pl.ds((c - _NBUF) * _G, _G)],
                osems[b]).wait()

        gates = []
        for g in range(_G):
            node = c * _G + g
            rowi = g * _K + lax.iota(_I32, 16)
            qs = plsc.load_gather(rowsq_v.at[b], [rowi, rel_v[node, :]])
            s = pg_v[node, :] + qs
            gates.append((1.0 / _K) / (1.0 + jnp.exp(-s)))
        for g in range(_G):
            gate = gates[g]
            z = jnp.zeros((16,), _F32)
            acc = [z, z, z, z]
            for k in range(_K):
                gk = gate[k]
                acc[k % 2] = acc[k % 2] + gk * rowsx_v[b, g * _K + k, 0:16]
                acc[2 + k % 2] = (acc[2 + k % 2]
                                  + gk * rowsx_v[b, g * _K + k, 16:32])
            out_v[b, g, 0:16] = acc[0] + acc[1]
            out_v[b, g, 16:32] = acc[2] + acc[3]
        pltpu.async_copy(out_v.at[b], out_hbm.at[w, pl.ds(c * _G, _G)],
                         osems[b])

        @pl.when(c + _NBUF < _CH)
        def _():
            pltpu.async_copy(tx_hbm.at[adj_v.at[c + _NBUF]], rowsx_v.at[b],
                             gxsems[b])
            pltpu.async_copy(tq_hbm.at[adj_v.at[c + _NBUF]], rowsq_v.at[b],
                             gqsems[b])

    def body(i, carry):
        c0 = i * _NBUF
        for b in range(_NBUF):
            chunk(c0 + b, b)
        return carry

    lax.fori_loop(0, _CH // _NBUF, body, 0)
    for b in range(_NBUF):
        c_last = _CH - _NBUF + b
        pltpu.make_async_copy(
            out_v.at[b], out_hbm.at[w, pl.ds(c_last * _G, _G)],
            osems[b]).wait()


@functools.cache
def _sc_agg():
    return pl.kernel(
        _sc_agg_body,
        out_type=jax.ShapeDtypeStruct((_NW, _CPW, 32), _F32),
        mesh=_mesh(),
        compiler_params=pltpu.CompilerParams(needs_layout_passes=False,
                                             use_tc_tiling_on_sc=False),
        scratch_types=[
            pltpu.VMEM((_CH, _G * _K), _I32),
            pltpu.VMEM((_CPW, _K), _I32),
            pltpu.VMEM((_CPW, _K), _F32),
            pltpu.VMEM((_NBUF, _G * _K, 32), _F32),
            pltpu.VMEM((_NBUF, _G * _K, _R), _F32),
            pltpu.VMEM((_NBUF, _G, 32), _F32),
        ] + [pltpu.SemaphoreType.DMA] * (3 * _NBUF),
    )


def _sc_pairs_body(fin_hbm, ui_hbm, ii_hbm, uo_hbm, io_hbm,
                   ui_v, ii_v, ur_v, ir_v, s1, s2):
    w = _wid()
    pltpu.sync_copy(ui_hbm.at[w], ui_v)
    pltpu.sync_copy(ii_hbm.at[w], ii_v)
    cu = pltpu.async_copy(fin_hbm.at[ui_v], ur_v, s1)
    ci = pltpu.async_copy(fin_hbm.at[ii_v], ir_v, s2)
    cu.wait()
    ci.wait()
    pltpu.sync_copy(ur_v, uo_hbm.at[w])
    pltpu.sync_copy(ir_v, io_hbm.at[w])


_PPW = _B // _NW


@functools.cache
def _sc_pairs():
    return pl.kernel(
        _sc_pairs_body,
        out_type=(jax.ShapeDtypeStruct((_NW, _PPW, _TW), _F32),
                  jax.ShapeDtypeStruct((_NW, _PPW, _TW), _F32)),
        mesh=_mesh(),
        compiler_params=pltpu.CompilerParams(needs_layout_passes=False,
                                             use_tc_tiling_on_sc=False),
        scratch_types=[
            pltpu.VMEM((_PPW,), _I32),
            pltpu.VMEM((_PPW,), _I32),
            pltpu.VMEM((_PPW, _TW), _F32),
            pltpu.VMEM((_PPW, _TW), _F32),
            pltpu.SemaphoreType.DMA,
            pltpu.SemaphoreType.DMA,
        ],
    )


def kernel(users, items, adj_list, adj_relation, entity_emb, user_emb,
           We1, be1, We2, be2, Wd1, bd1, Wd2, bd2,
           Wr1, lw1, lb1, Wr2, lw2, lb2):
    pad = _NP - _N
    emb = jnp.concatenate([entity_emb, user_emb], axis=0)
    emb_p = jnp.pad(emb, ((0, pad), (0, 0)))
    adj_p = jnp.pad(adj_list.astype(_I32), ((0, pad), (0, 0)))
    rel_p = jnp.pad(adj_relation.astype(_I32), ((0, pad), (0, 0)))

    tx1, tq1, pg1, enc, sq = pl.pallas_call(
        _tc_encode_body,
        grid=(_NBLK,),
        in_specs=[
            _rows(64), _rows(_K),
            _full((64, 128)), _full((1, 128)), _full((128, 32)),
            _full((1, 32)), _full((32, 128)), _full((1, 128)),
            _full((128, 64)), _full((1, 64)), _full((_R, 64, 32)),
        ],
        out_specs=[
            _rows(32), _rows(_R), _rows(_K), _rows(32),
            pl.BlockSpec((1, 1), lambda i: (0, 0),
                         memory_space=pltpu.SMEM),
        ],
        out_shape=[
            jax.ShapeDtypeStruct((_NP, 32), _F32),
            jax.ShapeDtypeStruct((_NP, _R), _F32),
            jax.ShapeDtypeStruct((_NP, _K), _F32),
            jax.ShapeDtypeStruct((_NP, 32), _F32),
            jax.ShapeDtypeStruct((1, 1), _F32),
        ],
    )(emb_p, rel_p, We1, be1.reshape(1, -1), We2, be2.reshape(1, -1),
      Wd1, bd1.reshape(1, -1), Wd2, bd2.reshape(1, -1), Wr1)
    ae_loss = sq[0, 0] / (_N * 64)

    adj_r = adj_p.reshape(_NW, _CH, _G * _K)
    rel_r = rel_p.reshape(_NW, _CPW, _K)

    agg1 = _sc_agg()(tx1, tq1, adj_r, rel_r,
                     pg1.reshape(_NW, _CPW, _K)).reshape(_NP, 32)

    x1, tx2, tq2, pg2 = pl.pallas_call(
        _tc_combine_body,
        grid=(_NBLK,),
        in_specs=[
            _rows(32), _rows(32), _rows(_K),
            _full((64, 32)), _full((1, 32)), _full((_R, 64, 32)),
        ],
        out_specs=[_rows(32), _rows(32), _rows(_R), _rows(_K)],
        out_shape=[
            jax.ShapeDtypeStruct((_NP, 32), _F32),
            jax.ShapeDtypeStruct((_NP, 32), _F32),
            jax.ShapeDtypeStruct((_NP, _R), _F32),
            jax.ShapeDtypeStruct((_NP, _K), _F32),
        ],
    )(enc, agg1, rel_p, lw1, lb1.reshape(1, -1), Wr2)

    agg2 = _sc_agg()(tx2, tq2, adj_r, rel_r,
                     pg2.reshape(_NW, _CPW, _K)).reshape(_NP, 32)

    fin = pl.pallas_call(
        _tc_final_body,
        grid=(_NBLK,),
        in_specs=[_rows(32), _rows(32), _full((64, 16)), _full((1, 16))],
        out_specs=[_rows(_TW)],
        out_shape=[jax.ShapeDtypeStruct((_NP, _TW), _F32)],
    )(x1, agg2, lw2, lb2.reshape(1, -1))[0]

    uidx = (users.astype(_I32) + _N_ENT).reshape(_NW, _PPW)
    iidx = items.astype(_I32).reshape(_NW, _PPW)
    urows, irows = _sc_pairs()(fin, uidx, iidx)

    scores = pl.pallas_call(
        _tc_score_body,
        grid=(_B // _SBLK,),
        in_specs=[pl.BlockSpec((_SBLK, _TW), lambda i: (i, 0)),
                  pl.BlockSpec((_SBLK, _TW), lambda i: (i, 0))],
        out_specs=[pl.BlockSpec((_SBLK, 1), lambda i: (i, 0))],
        out_shape=[jax.ShapeDtypeStruct((_B, 1), _F32)],
    )(urows.reshape(_B, _TW), irows.reshape(_B, _TW))[0]

    return scores.reshape(_B), ae_loss
